# baseline (speedup 1.0000x reference)
import functools

import jax
import jax.numpy as jnp
from jax import lax
from jax.experimental import pallas as pl
from jax.experimental.pallas import tpu as pltpu
from jax.experimental.pallas import tpu_sc as plsc

D_MODEL = 2048
SCALE = float(D_MODEL) ** 0.5
NC, NS, L = 2, 16, 16
NW = NC * NS
B_TOTAL = 4 * 4096
B_PER_W = B_TOTAL // NW
CG = 16
CS = 8
N_G = B_PER_W // CG
N_ROUNDS = N_G // 2


@functools.cache
def _make_emb():
    mesh = plsc.VectorSubcoreMesh(
        core_axis_name="c", subcore_axis_name="s",
        num_cores=NC, num_subcores=NS)

    @functools.partial(
        pl.kernel,
        out_type=jax.ShapeDtypeStruct((B_TOTAL, D_MODEL), jnp.float32),
        mesh=mesh,
        scratch_types=(
            [pltpu.VMEM((B_PER_W,), jnp.int32)]
            + [pltpu.VMEM((CG, D_MODEL), jnp.float32)] * 2
            + [pltpu.VMEM((CS, D_MODEL), jnp.float32)] * 2
            + [pltpu.SemaphoreType.DMA] * 4
        ),
    )
    def emb(idx_hbm, table_hbm, out_hbm, idx_v,
            gb0, gb1, sb0, sb1, sem_g0, sem_g1, sem_s0, sem_s1):
        wid = lax.axis_index("s") * NC + lax.axis_index("c")
        base = wid * B_PER_W
        pltpu.sync_copy(idx_hbm.at[pl.ds(base, B_PER_W)], idx_v)

        gbufs = ((gb0, sem_g0), (gb1, sem_g1))
        sbufs = ((sb0, sem_s0), (sb1, sem_s1))

        def gather(gb, sem, G):
            return pltpu.make_async_copy(
                table_hbm.at[idx_v.at[pl.ds(G * CG, CG)]], gb, sem)

        def scatter(sb, sem, h):
            return pltpu.make_async_copy(
                sb, out_hbm.at[pl.ds(base + h * CS, CS)], sem)

        def scale_half(gb, half, sb):
            for r in range(CS):
                @plsc.parallel_loop(0, D_MODEL // L, unroll=8)
                def _(i):
                    sl = pl.ds(i * L, L)
                    sb[r, sl] = gb[half * CS + r, sl] * SCALE

        gather(gb0, sem_g0, 0).start()
        gather(gb1, sem_g1, 1).start()

        def round_body(p, carry):
            for a in range(2):
                G = 2 * p + a
                gb, sg = gbufs[a]
                gather(gb, sg, 0).wait()
                for half in range(2):
                    sb, ss = sbufs[half]
                    if a == 0:
                        @pl.when(p > 0)
                        def _():
                            scatter(sb, ss, 0).wait()
                    else:
                        scatter(sb, ss, 0).wait()
                    scale_half(gb, half, sb)
                    scatter(sb, ss, 2 * G + half).start()
                @pl.when(p < N_ROUNDS - 1)
                def _():
                    gather(gb, sg, G + 2).start()
            return carry

        lax.fori_loop(0, N_ROUNDS, round_body, None)
        scatter(sb0, sem_s0, 0).wait()
        scatter(sb1, sem_s1, 0).wait()

    return emb


def kernel(x, embedding_table):
    b, s = x.shape
    x_flat = x.reshape(-1).astype(jnp.int32)
    out = _make_emb()(x_flat, embedding_table)
    return out.reshape(b, s, D_MODEL)
